# Initial kernel scaffold; baseline (speedup 1.0000x reference)
import jax
import jax.numpy as jnp
from jax.experimental import pallas as pl


def kernel(pc1, pc2, feat1, feat2, wn1_w0, wn1_b0, wn1_w1, wn1_b1, wn1_w2, wn1_b2, lin1_w, lin1_b, wn2_w0, wn2_b0, wn2_w1, wn2_b1, wn2_w2, wn2_b2, lin2_w, lin2_b):
    raise NotImplementedError("write your pallas kernel here")


# speedup vs baseline: 11.9460x; 11.9460x over previous
import functools

import jax
import jax.numpy as jnp
from jax import lax
from jax.experimental import pallas as pl
from jax.experimental.pallas import tpu as pltpu
from jax.experimental.pallas import tpu_sc as plsc

N = 4096
K = 16
DT = 48
NQ = 256
NBQ = 256
SC_CHUNK = 128


def _topk_body(q_ref, k_ref, idx_ref):
    p = pl.program_id(0)
    q = q_ref[0]
    kx = k_ref[0]
    qk = lax.dot_general(q, kx, (((1,), (1,)), ((), ())),
                         preferred_element_type=jnp.float32)
    d = -2.0 * qk
    d = d + jnp.sum(q * q, axis=1, keepdims=True)
    d = d + jnp.sum(kx * kx, axis=1)[None, :]
    iota = lax.broadcasted_iota(jnp.int32, (1, N), 1)
    cols = []
    for _ in range(K):
        m = jnp.min(d, axis=1, keepdims=True)
        cand = jnp.where(d <= m, iota, jnp.int32(N))
        idx = jnp.min(cand, axis=1, keepdims=True)
        cols.append(idx)
        d = jnp.where(iota == idx, jnp.float32(jnp.inf), d)
    idx_ref[0] = jnp.concatenate(cols, axis=1) + p * N


def _run_topk(queries, keys, num_pairs):
    grid = (num_pairs, N // NQ)
    return pl.pallas_call(
        _topk_body,
        grid=grid,
        in_specs=[
            pl.BlockSpec((1, NQ, 3), lambda p, i: (p, i, 0)),
            pl.BlockSpec((1, N, 3), lambda p, i: (p, 0, 0)),
        ],
        out_specs=pl.BlockSpec((1, NQ, K), lambda p, i: (p, i, 0)),
        out_shape=jax.ShapeDtypeStruct((num_pairs, N, K), jnp.int32),
    )(queries, keys)


def _sc_gather_body(nchunks, rpw, table_hbm, idx_hbm, out_hbm,
                    idx_v, buf_v, sem):
    wid = lax.axis_index("s") * 2 + lax.axis_index("c")
    base = wid * rpw

    def step(i, carry):
        off = base + i * SC_CHUNK
        pltpu.sync_copy(idx_hbm.at[pl.ds(off, SC_CHUNK)], idx_v)
        pltpu.async_copy(table_hbm.at[idx_v], buf_v, sem).wait()
        pltpu.sync_copy(buf_v, out_hbm.at[pl.ds(off, SC_CHUNK)])
        return carry

    lax.fori_loop(0, nchunks, step, 0)


def _run_sc_gather(table, idx_flat):
    num_rows = idx_flat.shape[0]
    rpw = num_rows // 32
    nchunks = rpw // SC_CHUNK
    mesh = plsc.VectorSubcoreMesh(core_axis_name="c", subcore_axis_name="s")
    kern = pl.kernel(
        functools.partial(_sc_gather_body, nchunks, rpw),
        out_type=jax.ShapeDtypeStruct((num_rows, DT), jnp.float32),
        mesh=mesh,
        scratch_types=[
            pltpu.VMEM((SC_CHUNK,), jnp.int32),
            pltpu.VMEM((SC_CHUNK, DT), jnp.float32),
            pltpu.SemaphoreType.DMA,
        ],
        compiler_params=pltpu.CompilerParams(use_tc_tiling_on_sc=False),
    )
    return kern(table, idx_flat)


def _agg_body(g_ref, qrep_ref, p1_ref, w0t_ref, b0_ref, w1t_ref, b1_ref,
              w2t_ref, b2_ref, lwt_ref, lb_ref, out_ref):
    R = NBQ * K
    f32 = jnp.float32

    def mm(a, b):
        return lax.dot_general(a, b, (((1,), (0,)), ((), ())),
                               preferred_element_type=f32)

    dir_ = g_ref[:, 0:3] - qrep_ref[...]
    h = jnp.maximum(mm(dir_, w0t_ref[...]) + b0_ref[...], 0.0)
    h = jnp.maximum(mm(h, w1t_ref[...]) + b1_ref[...], 0.0)
    w = jnp.maximum(mm(h, w2t_ref[...]) + b2_ref[...], 0.0)

    j32 = lax.broadcasted_iota(jnp.int32, (32, 512), 1)
    r32 = lax.broadcasted_iota(jnp.int32, (32, 512), 0)
    rep = jnp.where((j32 >> 4) == r32, f32(1.0), f32(0.0))
    j16 = lax.broadcasted_iota(jnp.int32, (16, 512), 1)
    r16 = lax.broadcasted_iota(jnp.int32, (16, 512), 0)
    til = jnp.where((j16 & 15) == r16, f32(1.0), f32(0.0))

    s = jnp.sum(w.reshape(NBQ, K, 16), axis=1)
    out1 = mm(p1_ref[...], rep) * mm(s, til)

    feats = g_ref[:, 3:35]
    pp = mm(feats, rep) * mm(w, til)
    out2 = jnp.sum(pp.reshape(NBQ, K, 512), axis=1)

    lwt = lwt_ref[...]
    res = mm(out1, lwt[0:512]) + mm(out2, lwt[512:1024]) + lb_ref[...]
    out_ref[...] = jnp.where(res >= 0.0, res, 0.1 * res)


def _run_agg(g, qrep, p1, wn, lwt, lb, num_pairs):
    w0t, b0, w1t, b1, w2t, b2 = wn
    nblocks = N // NBQ
    grid = (num_pairs * nblocks,)
    R = NBQ * K

    def rows(i):
        return (i, 0)

    def const(i):
        return (0, 0)

    return pl.pallas_call(
        _agg_body,
        grid=grid,
        in_specs=[
            pl.BlockSpec((R, DT), rows),
            pl.BlockSpec((R, 3), rows),
            pl.BlockSpec((NBQ, 32), rows),
            pl.BlockSpec((3, 8), const),
            pl.BlockSpec((1, 8), const),
            pl.BlockSpec((8, 8), const),
            pl.BlockSpec((1, 8), const),
            pl.BlockSpec((8, 16), const),
            pl.BlockSpec((1, 16), const),
            pl.BlockSpec((1024, 32), const),
            pl.BlockSpec((1, 32), const),
        ],
        out_specs=pl.BlockSpec((NBQ, 32), rows),
        out_shape=jax.ShapeDtypeStruct((num_pairs * N, 32), jnp.float32),
    )(g, qrep, p1, w0t, b0, w1t, b1, w2t, b2, lwt, lb)


def kernel(pc1, pc2, feat1, feat2, wn1_w0, wn1_b0, wn1_w1, wn1_b1, wn1_w2,
           wn1_b2, lin1_w, lin1_b, wn2_w0, wn2_b0, wn2_w1, wn2_b1, wn2_w2,
           wn2_b2, lin2_w, lin2_b):
    x1 = jnp.transpose(pc1, (0, 2, 1))
    x2 = jnp.transpose(pc2, (0, 2, 1))
    f1 = jnp.transpose(feat1, (0, 2, 1))
    f2 = jnp.transpose(feat2, (0, 2, 1))

    queries = jnp.concatenate([x1, x2], axis=0)
    keyss = jnp.concatenate([x2, x1], axis=0)
    knn = _run_topk(queries, keyss, 4)
    idx_flat = knn.reshape(4 * N * K)

    pad = ((0, 0), (0, DT - 35))
    tbl12 = jnp.pad(
        jnp.concatenate([
            jnp.concatenate([x2, f2], axis=-1).reshape(2 * N, 35),
            jnp.concatenate([x1, f1], axis=-1).reshape(2 * N, 35),
        ], axis=0), pad)
    g12 = _run_sc_gather(tbl12, idx_flat)

    qflat = queries.reshape(4 * N, 3)
    qrep = jnp.repeat(qflat, K, axis=0)
    p1s = jnp.concatenate([f1, f2], axis=0).reshape(4 * N, 32)

    wn1 = (wn1_w0.T, wn1_b0[None, :], wn1_w1.T, wn1_b1[None, :],
           wn1_w2.T, wn1_b2[None, :])
    wn2 = (wn2_w0.T, wn2_b0[None, :], wn2_w1.T, wn2_b1[None, :],
           wn2_w2.T, wn2_b2[None, :])

    out12 = _run_agg(g12, qrep, p1s, wn1, lin1_w.T, lin1_b[None, :], 4)
    f1n = out12[:2 * N]
    f2n = out12[2 * N:]

    tbl3 = jnp.pad(
        jnp.concatenate([x2.reshape(2 * N, 3), f2n], axis=-1), pad)
    g3 = _run_sc_gather(tbl3, idx_flat[:2 * N * K])
    out3 = _run_agg(g3, qrep[:2 * N * K], f1n, wn2, lin2_w.T,
                    lin2_b[None, :], 2)

    feat1_new = jnp.transpose(f1n.reshape(2, N, 32), (0, 2, 1))
    feat2_new = jnp.transpose(f2n.reshape(2, N, 32), (0, 2, 1))
    feat1_final = jnp.transpose(out3.reshape(2, N, 32), (0, 2, 1))
    return (feat1_new, feat2_new, feat1_final)
